# Initial kernel scaffold; baseline (speedup 1.0000x reference)
import jax
import jax.numpy as jnp
from jax.experimental import pallas as pl


def kernel(x, lin_w, lin_b, tables, w1, b1, w2, b2, w3, b3):
    raise NotImplementedError("write your pallas kernel here")


# speedup vs baseline: 6.9244x; 6.9244x over previous
import functools

import jax
import jax.numpy as jnp
import numpy as np
from jax import lax
from jax.experimental import pallas as pl
from jax.experimental.pallas import tpu as pltpu
from jax.experimental.pallas import tpu_sc as plsc

F = 26
V = 1000
TOT = F * V
D = 16
B = 4096
NP0 = F * (F - 1) // 2
NPP = 328
RD = NPP * D // 128
FP = 32
LD = FP * D // 128
ZROW = TOT
_PI = np.array([i for i in range(F - 1) for j in range(i + 1, F)], dtype=np.int32)
_PJ = np.array([j for i in range(F - 1) for j in range(i + 1, F)], dtype=np.int32)

NC, NS = 2, 16
NW = NC * NS
PROWS = 2 * B * NPP
PRPW = PROWS // NW
PCHUNK = 2624
PNCHUNK = PRPW // PCHUNK
LROWS = B * FP
LRPW = LROWS // NW
LCHUNK = 2048
LNCHUNK = LRPW // LCHUNK


def _sc_gather(tflat, linpad, idxp, idxl):
    @functools.partial(
        pl.kernel,
        out_type=(
            jax.ShapeDtypeStruct((PROWS, D), jnp.float32),
            jax.ShapeDtypeStruct((LROWS, D), jnp.float32),
        ),
        mesh=plsc.VectorSubcoreMesh(core_axis_name="c", subcore_axis_name="s",
                                    num_cores=NC, num_subcores=NS),
        compiler_params=pltpu.CompilerParams(use_tc_tiling_on_sc=False),
        scratch_types=[
            pltpu.VMEM((PCHUNK,), jnp.int32),
            pltpu.VMEM((PCHUNK, D), jnp.float32),
            pltpu.VMEM((LCHUNK,), jnp.int32),
            pltpu.VMEM((LCHUNK, D), jnp.float32),
            pltpu.SemaphoreType.DMA,
        ],
    )
    def body(tflat_hbm, linpad_hbm, idxp_hbm, idxl_hbm, pall_hbm, plin_hbm,
             idx_v, rows_v, idxl_v, rowsl_v, sem):
        wid = lax.axis_index("s") * NC + lax.axis_index("c")
        base0 = wid * PRPW
        for c in range(PNCHUNK):
            base = base0 + c * PCHUNK
            pltpu.sync_copy(idxp_hbm.at[pl.ds(base, PCHUNK)], idx_v)
            pltpu.async_copy(tflat_hbm.at[idx_v], rows_v, sem).wait()
            pltpu.sync_copy(rows_v, pall_hbm.at[pl.ds(base, PCHUNK)])
        lbase0 = wid * LRPW
        for c in range(LNCHUNK):
            base = lbase0 + c * LCHUNK
            pltpu.sync_copy(idxl_hbm.at[pl.ds(base, LCHUNK)], idxl_v)
            pltpu.async_copy(linpad_hbm.at[idxl_v], rowsl_v, sem).wait()
            pltpu.sync_copy(rowsl_v, plin_hbm.at[pl.ds(base, LCHUNK)])

    return body(tflat, linpad, idxp, idxl)


def _tc_mlp(p1d, p2d, plind, w1b, b1b, w2b, b2b, w3b, b3b):
    BB = 128
    NB = B // BB

    def body(p1_ref, p2_ref, plin_ref, w1_ref, b1_ref, w2_ref, b2_ref,
             w3_ref, b3_ref, s1_ref, s2_ref):
        ix = p1_ref[...] * p2_ref[...]
        ix3 = ix.reshape(BB, RD, 128)
        ii = lax.broadcasted_iota(jnp.int32, (BB, RD, 128), 1)
        jj = lax.broadcasted_iota(jnp.int32, (BB, RD, 128), 2)
        dummy = (ii == RD - 1) & (jj >= 128 - (NPP - NP0) * D)
        ixm = jnp.where(dummy, 0.0, ix3)
        ffm = jnp.sum(jnp.sum(ixm, axis=1), axis=1, keepdims=True)
        lin3 = plin_ref[...].reshape(BB, LD, 128)
        lin = jnp.sum(jnp.sum(lin3, axis=1), axis=1, keepdims=True)
        xmat = ixm.reshape(BB * RD, 128)
        h1 = jax.nn.relu(
            jnp.dot(xmat, w1_ref[...], preferred_element_type=jnp.float32)
            + b1_ref[...])
        h2 = jax.nn.relu(
            jnp.dot(h1, w2_ref[...], preferred_element_type=jnp.float32)
            + b2_ref[...])
        fi = (jnp.dot(h2, w3_ref[...], preferred_element_type=jnp.float32)
              + b3_ref[...])
        fi3 = fi.reshape(BB, RD, 8)
        fisum = jnp.sum(jnp.sum(fi3, axis=1), axis=1, keepdims=True)
        s1_ref[...] = lin + ffm
        s2_ref[...] = fisum

    return pl.pallas_call(
        body,
        grid=(NB,),
        in_specs=[
            pl.BlockSpec((BB * RD, 128), lambda i: (i, 0)),
            pl.BlockSpec((BB * RD, 128), lambda i: (i, 0)),
            pl.BlockSpec((BB * LD, 128), lambda i: (i, 0)),
            pl.BlockSpec((128, 512), lambda i: (0, 0)),
            pl.BlockSpec((1, 512), lambda i: (0, 0)),
            pl.BlockSpec((512, 256), lambda i: (0, 0)),
            pl.BlockSpec((1, 256), lambda i: (0, 0)),
            pl.BlockSpec((256, 8), lambda i: (0, 0)),
            pl.BlockSpec((1, 8), lambda i: (0, 0)),
        ],
        out_specs=[
            pl.BlockSpec((BB, 1), lambda i: (i, 0)),
            pl.BlockSpec((BB, 1), lambda i: (i, 0)),
        ],
        out_shape=[
            jax.ShapeDtypeStruct((B, 1), jnp.float32),
            jax.ShapeDtypeStruct((B, 1), jnp.float32),
        ],
    )(p1d, p2d, plind, w1b, b1b, w2b, b2b, w3b, b3b)


def _tc_outer(s1, s2t):
    RB = 256

    def body(s1_ref, s2_ref, out_ref):
        out_ref[...] = jax.nn.sigmoid(s1_ref[...] + s2_ref[...])

    return pl.pallas_call(
        body,
        grid=(B // RB,),
        in_specs=[
            pl.BlockSpec((RB, 1), lambda i: (i, 0)),
            pl.BlockSpec((1, B), lambda i: (0, 0)),
        ],
        out_specs=pl.BlockSpec((RB, B), lambda i: (i, 0)),
        out_shape=jax.ShapeDtypeStruct((B, B), jnp.float32),
    )(s1, s2t)


def kernel(x, lin_w, lin_b, tables, w1, b1, w2, b2, w3, b3):
    x = x.astype(jnp.int32)
    offs = (jnp.arange(F, dtype=jnp.int32) * V)[None, :]
    xo = x + offs
    pi = jnp.asarray(_PI)
    pj = jnp.asarray(_PJ)
    idx1 = pj[None, :] * TOT + jnp.take(xo, pi, axis=1)
    idx2 = pi[None, :] * TOT + jnp.take(xo, pj, axis=1)
    pad = jnp.zeros((B, NPP - NP0), jnp.int32)
    idxp = jnp.concatenate([
        jnp.concatenate([idx1, pad], axis=1).reshape(-1),
        jnp.concatenate([idx2, pad], axis=1).reshape(-1),
    ])
    idxl = jnp.concatenate(
        [xo, jnp.full((B, FP - F), ZROW, jnp.int32)], axis=1).reshape(-1)
    tflat = tables.reshape(F * TOT, D)
    linpad = jnp.zeros((TOT + 8, D), jnp.float32).at[:TOT, 0].set(
        lin_w.reshape(TOT))

    pall, plin = _sc_gather(tflat, linpad, idxp, idxl)

    pd = pall.reshape(2, B * RD, 128)
    plind = plin.reshape(B * LD, 128)

    eye8 = jnp.eye(8, dtype=jnp.float32)
    w1b = jnp.kron(eye8, w1.T)
    w2b = jnp.kron(eye8, w2.T)
    w3b = jnp.kron(eye8, w3.T)
    b1b = jnp.tile(b1, 8)[None, :]
    b2b = jnp.tile(b2, 8)[None, :]
    b3b = jnp.tile(b3, 8)[None, :]

    s1, s2 = _tc_mlp(pd[0], pd[1], plind, w1b, b1b, w2b, b2b, w3b, b3b)

    s1 = s1 + lin_b[0]
    cpad = (w3 @ jax.nn.relu(w2 @ jax.nn.relu(b1) + b2) + b3)[0]
    s2 = s2 - (NPP - NP0) * cpad

    return _tc_outer(s1, s2.reshape(1, B))
